# Initial kernel scaffold; baseline (speedup 1.0000x reference)
import functools

import jax
import jax.numpy as jnp
from jax import lax
from jax.experimental import pallas as pl
from jax.experimental.pallas import tpu as pltpu
from jax.experimental.pallas import tpu_sc as plsc

D_MODEL = 768
D_FF = 1024
N_EXPERTS = 64
CAPACITY = 80
AUX_COEF = 0.01
ZLOSS_COEF = 0.001
EPS = 1e-9

N_TOKENS = 4096
BLK_T = 512
N_BLK = N_TOKENS // BLK_T
DUMP = N_EXPERTS * CAPACITY
SLOTS = (N_EXPERTS + 1) * CAPACITY

SC_NC = 2
SC_NS = 16
SC_NW = SC_NC * SC_NS
TOK_PER_W = N_TOKENS // SC_NW


def _router_body(x_ref, rw_ref, pos_ref, scale_ref, load_ref, imp_ref,
                 aux_ref, carry, imp_acc, z2_acc):
    i = pl.program_id(0)

    @pl.when(i == 0)
    def _init():
        carry[...] = jnp.zeros_like(carry)
        imp_acc[...] = jnp.zeros_like(imp_acc)
        z2_acc[0, 0] = 0.0

    x = x_ref[...]
    logits = lax.dot_general(
        x, rw_ref[...], (((1,), (1,)), ((), ())),
        preferred_element_type=jnp.float32,
        precision=lax.Precision.HIGHEST)
    m = jnp.max(logits, axis=1, keepdims=True)
    ex = jnp.exp(logits - m)
    s = jnp.sum(ex, axis=1, keepdims=True)
    probs = ex / s
    v = jnp.max(probs, axis=1, keepdims=True)
    iota_e = lax.broadcasted_iota(jnp.int32, (BLK_T, N_EXPERTS), 1)
    idx = jnp.min(jnp.where(probs == v, iota_e, N_EXPERTS), axis=1,
                  keepdims=True)
    oh = (iota_e == idx).astype(jnp.float32)

    r_i = lax.broadcasted_iota(jnp.int32, (BLK_T, BLK_T), 0)
    c_i = lax.broadcasted_iota(jnp.int32, (BLK_T, BLK_T), 1)
    tril = (c_i < r_i).astype(jnp.float32)
    prev = lax.dot_general(
        tril, oh, (((1,), (0,)), ((), ())),
        preferred_element_type=jnp.float32,
        precision=lax.Precision.HIGHEST)
    rank_f = jnp.sum(oh * (prev + carry[0:1, :]), axis=1, keepdims=True)
    rank = rank_f.astype(jnp.int32)
    keep = rank < CAPACITY
    pos_ref[...] = jnp.where(keep, idx * CAPACITY + rank, DUMP)
    gate = v / (v + EPS)
    scale = jnp.where(keep, gate, 0.0)
    scale_ref[...] = jnp.broadcast_to(scale, (BLK_T, 16))

    carry[...] = carry[...] + jnp.sum(oh, axis=0, keepdims=True)
    imp_acc[...] = imp_acc[...] + jnp.sum(probs, axis=0, keepdims=True)
    z = m[:, 0] + jnp.log(s[:, 0])
    z2_acc[0, 0] = z2_acc[0, 0] + jnp.sum(z * z)

    @pl.when(i == N_BLK - 1)
    def _fin():
        cnt = carry[...]
        load = jnp.minimum(cnt, float(CAPACITY))
        load_ref[...] = load.astype(jnp.int32)
        imp = imp_acc[...]
        imp_ref[...] = imp
        imp_norm = imp[0:1, :] / (jnp.sum(imp[0:1, :]) + EPS)
        load_norm = load[0:1, :] / (jnp.sum(load[0:1, :]) + EPS)
        balance = N_EXPERTS * jnp.sum(imp_norm * load_norm)
        zloss = z2_acc[0, 0] / N_TOKENS
        aux = AUX_COEF * balance + ZLOSS_COEF * zloss
        aux_ref[...] = jnp.full((8, 128), aux, dtype=jnp.float32)


def _router(x_flat, router_w):
    return pl.pallas_call(
        _router_body,
        grid=(N_BLK,),
        in_specs=[
            pl.BlockSpec((BLK_T, D_MODEL), lambda i: (i, 0)),
            pl.BlockSpec((N_EXPERTS, D_MODEL), lambda i: (0, 0)),
        ],
        out_specs=[
            pl.BlockSpec((BLK_T, 1), lambda i: (i, 0)),
            pl.BlockSpec((BLK_T, 16), lambda i: (i, 0)),
            pl.BlockSpec((8, N_EXPERTS), lambda i: (0, 0)),
            pl.BlockSpec((8, N_EXPERTS), lambda i: (0, 0)),
            pl.BlockSpec((8, 128), lambda i: (0, 0)),
        ],
        out_shape=[
            jax.ShapeDtypeStruct((N_TOKENS, 1), jnp.int32),
            jax.ShapeDtypeStruct((N_TOKENS, 16), jnp.float32),
            jax.ShapeDtypeStruct((8, N_EXPERTS), jnp.int32),
            jax.ShapeDtypeStruct((8, N_EXPERTS), jnp.float32),
            jax.ShapeDtypeStruct((8, 128), jnp.float32),
        ],
        scratch_shapes=[
            pltpu.VMEM((8, N_EXPERTS), jnp.float32),
            pltpu.VMEM((8, N_EXPERTS), jnp.float32),
            pltpu.SMEM((1, 1), jnp.float32),
        ],
    )(x_flat, router_w)


def _dispatch(x_flat, pos, scale_exp):
    mesh = plsc.VectorSubcoreMesh(core_axis_name="c", subcore_axis_name="s")

    @functools.partial(
        pl.kernel, mesh=mesh,
        out_type=[
            jax.ShapeDtypeStruct((SLOTS, D_MODEL), jnp.float32),
            jax.ShapeDtypeStruct((SLOTS, 16), jnp.float32),
        ],
        scratch_types=[
            pltpu.VMEM((TOK_PER_W,), jnp.int32),
            pltpu.VMEM((TOK_PER_W, D_MODEL), jnp.float32),
            pltpu.VMEM((TOK_PER_W, 16), jnp.float32),
            pltpu.SemaphoreType.DMA,
            pltpu.SemaphoreType.DMA,
        ],
    )
    def disp(x_hbm, pos_hbm, se_hbm, xbuf_hbm, sbuf_hbm,
             idx_v, xrows_v, srows_v, sem1, sem2):
        wid = lax.axis_index("s") * SC_NC + lax.axis_index("c")
        base = wid * TOK_PER_W
        pltpu.sync_copy(pos_hbm.at[pl.ds(base, TOK_PER_W)], idx_v)
        pltpu.sync_copy(x_hbm.at[pl.ds(base, TOK_PER_W)], xrows_v)
        pltpu.sync_copy(se_hbm.at[pl.ds(base, TOK_PER_W)], srows_v)
        cp1 = pltpu.async_copy(xrows_v, xbuf_hbm.at[idx_v], sem1)
        cp2 = pltpu.async_copy(srows_v, sbuf_hbm.at[idx_v], sem2)
        cp1.wait()
        cp2.wait()

    return disp(x_flat, pos, scale_exp)


def _ffn_body(x_ref, w1_ref, b1_ref, w2_ref, b2_ref, s_ref, o_ref):
    xin = x_ref[...]
    h = jnp.dot(xin, w1_ref[0], preferred_element_type=jnp.float32,
                precision=lax.Precision.HIGHEST) + b1_ref[...]
    h = 0.5 * h * (1.0 + lax.erf(h * 0.7071067811865476))
    out = jnp.dot(h, w2_ref[0], preferred_element_type=jnp.float32,
                  precision=lax.Precision.HIGHEST) + b2_ref[...]
    s = s_ref[...][:, 0:1]
    o_ref[...] = jnp.where(s > 0.0, out * s, 0.0)


def _ffn(xbuf, sbuf, w1, b1, w2, b2):
    grid = N_EXPERTS + 1
    return pl.pallas_call(
        _ffn_body,
        grid=(grid,),
        in_specs=[
            pl.BlockSpec((CAPACITY, D_MODEL), lambda e: (e, 0)),
            pl.BlockSpec((1, D_MODEL, D_FF),
                         lambda e: (jnp.minimum(e, N_EXPERTS - 1), 0, 0)),
            pl.BlockSpec((1, D_FF),
                         lambda e: (jnp.minimum(e, N_EXPERTS - 1), 0)),
            pl.BlockSpec((1, D_FF, D_MODEL),
                         lambda e: (jnp.minimum(e, N_EXPERTS - 1), 0, 0)),
            pl.BlockSpec((1, D_MODEL),
                         lambda e: (jnp.minimum(e, N_EXPERTS - 1), 0)),
            pl.BlockSpec((CAPACITY, 16), lambda e: (e, 0)),
        ],
        out_specs=pl.BlockSpec((CAPACITY, D_MODEL), lambda e: (e, 0)),
        out_shape=jax.ShapeDtypeStruct((SLOTS, D_MODEL), jnp.float32),
    )(xbuf, w1, b1, w2, b2, sbuf)


def _combine(obuf, pos):
    mesh = plsc.VectorSubcoreMesh(core_axis_name="c", subcore_axis_name="s")

    @functools.partial(
        pl.kernel, mesh=mesh,
        out_type=jax.ShapeDtypeStruct((N_TOKENS, D_MODEL), jnp.float32),
        scratch_types=[
            pltpu.VMEM((TOK_PER_W,), jnp.int32),
            pltpu.VMEM((TOK_PER_W, D_MODEL), jnp.float32),
            pltpu.SemaphoreType.DMA,
        ],
    )
    def comb(obuf_hbm, pos_hbm, y_hbm, idx_v, rows_v, sem):
        wid = lax.axis_index("s") * SC_NC + lax.axis_index("c")
        base = wid * TOK_PER_W
        pltpu.sync_copy(pos_hbm.at[pl.ds(base, TOK_PER_W)], idx_v)
        pltpu.async_copy(obuf_hbm.at[idx_v], rows_v, sem).wait()
        pltpu.sync_copy(rows_v, y_hbm.at[pl.ds(base, TOK_PER_W)])

    return comb(obuf, pos)


def kernel(x, router_w, w1, b1, w2, b2):
    B, T, D = x.shape
    x_flat = x.reshape(B * T, D)
    pos2, scale_exp, load8, imp8, aux8 = _router(x_flat, router_w)
    pos = pos2.reshape(B * T)
    xbuf, sbuf = _dispatch(x_flat, pos, scale_exp)
    obuf = _ffn(xbuf, sbuf, w1, b1, w2, b2)
    y_flat = _combine(obuf, pos)
    y = y_flat.reshape(B, T, D)
    return (y, aux8[0, 0], load8[0], imp8[0])


# speedup vs baseline: 1.2103x; 1.2103x over previous
import functools

import jax
import jax.numpy as jnp
from jax import lax
from jax.experimental import pallas as pl
from jax.experimental.pallas import tpu as pltpu
from jax.experimental.pallas import tpu_sc as plsc

D_MODEL = 768
D_FF = 1024
N_EXPERTS = 64
CAPACITY = 80
AUX_COEF = 0.01
ZLOSS_COEF = 0.001
EPS = 1e-9

N_TOKENS = 4096
BLK_T = 512
N_BLK = N_TOKENS // BLK_T
DUMP = N_EXPERTS * CAPACITY
SLOTS = (N_EXPERTS + 1) * CAPACITY

SC_NC = 2
SC_NS = 16
SC_NW = SC_NC * SC_NS
TOK_PER_W = N_TOKENS // SC_NW


def _router_body(l_ref, pos_ref, scale_ref, load_ref, imp_ref,
                 aux_ref, carry, imp_acc, z2_acc):
    i = pl.program_id(0)

    @pl.when(i == 0)
    def _init():
        carry[...] = jnp.zeros_like(carry)
        imp_acc[...] = jnp.zeros_like(imp_acc)
        z2_acc[0, 0] = 0.0

    logits = l_ref[...]
    m = jnp.max(logits, axis=1, keepdims=True)
    ex = jnp.exp(logits - m)
    s = jnp.sum(ex, axis=1, keepdims=True)
    probs = ex / s
    v = jnp.max(probs, axis=1, keepdims=True)
    iota_e = lax.broadcasted_iota(jnp.int32, (BLK_T, N_EXPERTS), 1)
    idx = jnp.min(jnp.where(probs == v, iota_e, N_EXPERTS), axis=1,
                  keepdims=True)
    oh = (iota_e == idx).astype(jnp.float32)

    r_i = lax.broadcasted_iota(jnp.int32, (BLK_T, BLK_T), 0)
    c_i = lax.broadcasted_iota(jnp.int32, (BLK_T, BLK_T), 1)
    tril = (c_i < r_i).astype(jnp.float32)
    prev = lax.dot_general(
        tril, oh, (((1,), (0,)), ((), ())),
        preferred_element_type=jnp.float32,
        precision=lax.Precision.DEFAULT)
    rank_f = jnp.sum(oh * (prev + carry[0:1, :]), axis=1, keepdims=True)
    rank = rank_f.astype(jnp.int32)
    keep = rank < CAPACITY
    pos_ref[...] = jnp.where(keep, idx * CAPACITY + rank, DUMP)
    gate = v / (v + EPS)
    scale = jnp.where(keep, gate, 0.0)
    scale_ref[...] = jnp.broadcast_to(scale, (BLK_T, 128))

    carry[...] = carry[...] + jnp.sum(oh, axis=0, keepdims=True)
    imp_acc[...] = imp_acc[...] + jnp.sum(probs, axis=0, keepdims=True)
    z = m[:, 0] + jnp.log(s[:, 0])
    z2_acc[0, 0] = z2_acc[0, 0] + jnp.sum(z * z)

    @pl.when(i == N_BLK - 1)
    def _fin():
        cnt = carry[...]
        load = jnp.minimum(cnt, float(CAPACITY))
        load_ref[...] = load.astype(jnp.int32)
        imp = imp_acc[...]
        imp_ref[...] = imp
        imp_norm = imp[0:1, :] / (jnp.sum(imp[0:1, :]) + EPS)
        load_norm = load[0:1, :] / (jnp.sum(load[0:1, :]) + EPS)
        balance = N_EXPERTS * jnp.sum(imp_norm * load_norm)
        zloss = z2_acc[0, 0] / N_TOKENS
        aux = AUX_COEF * balance + ZLOSS_COEF * zloss
        aux_ref[...] = jnp.full((8, 128), aux, dtype=jnp.float32)


def _router(logits):
    return pl.pallas_call(
        _router_body,
        grid=(N_BLK,),
        in_specs=[
            pl.BlockSpec((BLK_T, N_EXPERTS), lambda i: (i, 0)),
        ],
        out_specs=[
            pl.BlockSpec((BLK_T, 1), lambda i: (i, 0)),
            pl.BlockSpec((BLK_T, 128), lambda i: (i, 0)),
            pl.BlockSpec((8, N_EXPERTS), lambda i: (0, 0)),
            pl.BlockSpec((8, N_EXPERTS), lambda i: (0, 0)),
            pl.BlockSpec((8, 128), lambda i: (0, 0)),
        ],
        out_shape=[
            jax.ShapeDtypeStruct((N_TOKENS, 1), jnp.int32),
            jax.ShapeDtypeStruct((N_TOKENS, 128), jnp.float32),
            jax.ShapeDtypeStruct((8, N_EXPERTS), jnp.int32),
            jax.ShapeDtypeStruct((8, N_EXPERTS), jnp.float32),
            jax.ShapeDtypeStruct((8, 128), jnp.float32),
        ],
        scratch_shapes=[
            pltpu.VMEM((8, N_EXPERTS), jnp.float32),
            pltpu.VMEM((8, N_EXPERTS), jnp.float32),
            pltpu.SMEM((1, 1), jnp.float32),
        ],
    )(logits)


def _dispatch(x_flat, pos, scale_exp):
    mesh = plsc.VectorSubcoreMesh(core_axis_name="c", subcore_axis_name="s")

    @functools.partial(
        pl.kernel, mesh=mesh,
        out_type=[
            jax.ShapeDtypeStruct((SLOTS, D_MODEL), jnp.float32),
            jax.ShapeDtypeStruct((SLOTS, 128), jnp.float32),
        ],
        scratch_types=[
            pltpu.VMEM((TOK_PER_W,), jnp.int32),
            pltpu.VMEM((TOK_PER_W, D_MODEL), jnp.float32),
            pltpu.VMEM((TOK_PER_W, 128), jnp.float32),
            pltpu.SemaphoreType.DMA,
            pltpu.SemaphoreType.DMA,
        ],
    )
    def disp(x_hbm, pos_hbm, se_hbm, xbuf_hbm, sbuf_hbm,
             idx_v, xrows_v, srows_v, sem1, sem2):
        wid = lax.axis_index("s") * SC_NC + lax.axis_index("c")
        base = wid * TOK_PER_W
        pltpu.sync_copy(pos_hbm.at[pl.ds(base, TOK_PER_W)], idx_v)
        pltpu.sync_copy(x_hbm.at[pl.ds(base, TOK_PER_W)], xrows_v)
        pltpu.sync_copy(se_hbm.at[pl.ds(base, TOK_PER_W)], srows_v)
        cp1 = pltpu.async_copy(xrows_v, xbuf_hbm.at[idx_v], sem1)
        cp2 = pltpu.async_copy(srows_v, sbuf_hbm.at[idx_v], sem2)
        cp1.wait()
        cp2.wait()

    return disp(x_flat, pos, scale_exp)


def _ffn_body(x_ref, w1_ref, b1_ref, w2_ref, b2_ref, s_ref, o_ref):
    xin = x_ref[...]
    h = jnp.dot(xin, w1_ref[0], preferred_element_type=jnp.float32,
                precision=lax.Precision.HIGHEST) + b1_ref[0]
    h = 0.5 * h * (1.0 + lax.erf(h * 0.7071067811865476))
    out = jnp.dot(h, w2_ref[0], preferred_element_type=jnp.float32,
                  precision=lax.Precision.HIGHEST) + b2_ref[0]
    s = s_ref[...][:, 0:1]
    o_ref[...] = jnp.where(s > 0.0, out * s, 0.0)


def _ffn(xbuf, sbuf, w1, b1, w2, b2):
    grid = N_EXPERTS + 1
    return pl.pallas_call(
        _ffn_body,
        grid=(grid,),
        in_specs=[
            pl.BlockSpec((CAPACITY, D_MODEL), lambda e: (e, 0)),
            pl.BlockSpec((1, D_MODEL, D_FF),
                         lambda e: (jnp.minimum(e, N_EXPERTS - 1), 0, 0)),
            pl.BlockSpec((1, 1, D_FF),
                         lambda e: (jnp.minimum(e, N_EXPERTS - 1), 0, 0)),
            pl.BlockSpec((1, D_FF, D_MODEL),
                         lambda e: (jnp.minimum(e, N_EXPERTS - 1), 0, 0)),
            pl.BlockSpec((1, 1, D_MODEL),
                         lambda e: (jnp.minimum(e, N_EXPERTS - 1), 0, 0)),
            pl.BlockSpec((CAPACITY, 128), lambda e: (e, 0)),
        ],
        out_specs=pl.BlockSpec((CAPACITY, D_MODEL), lambda e: (e, 0)),
        out_shape=jax.ShapeDtypeStruct((SLOTS, D_MODEL), jnp.float32),
    )(xbuf, w1, b1.reshape(N_EXPERTS, 1, D_FF), w2,
      b2.reshape(N_EXPERTS, 1, D_MODEL), sbuf)


def _combine(obuf, pos):
    mesh = plsc.VectorSubcoreMesh(core_axis_name="c", subcore_axis_name="s")

    @functools.partial(
        pl.kernel, mesh=mesh,
        out_type=jax.ShapeDtypeStruct((N_TOKENS, D_MODEL), jnp.float32),
        scratch_types=[
            pltpu.VMEM((TOK_PER_W,), jnp.int32),
            pltpu.VMEM((TOK_PER_W, D_MODEL), jnp.float32),
            pltpu.SemaphoreType.DMA,
        ],
    )
    def comb(obuf_hbm, pos_hbm, y_hbm, idx_v, rows_v, sem):
        wid = lax.axis_index("s") * SC_NC + lax.axis_index("c")
        base = wid * TOK_PER_W
        pltpu.sync_copy(pos_hbm.at[pl.ds(base, TOK_PER_W)], idx_v)
        pltpu.async_copy(obuf_hbm.at[idx_v], rows_v, sem).wait()
        pltpu.sync_copy(rows_v, y_hbm.at[pl.ds(base, TOK_PER_W)])

    return comb(obuf, pos)


def kernel(x, router_w, w1, b1, w2, b2):
    B, T, D = x.shape
    x_flat = x.reshape(B * T, D)
    logits = x_flat @ router_w.T
    pos2, scale_exp, load8, imp8, aux8 = _router(logits)
    pos = pos2.reshape(B * T)
    xbuf, sbuf = _dispatch(x_flat, pos, scale_exp)
    obuf = _ffn(xbuf, sbuf, w1, b1, w2, b2)
    y_flat = _combine(obuf, pos)
    y = y_flat.reshape(B, T, D)
    return (y, aux8[0, 0], load8[0], imp8[0])


# speedup vs baseline: 1.9572x; 1.6172x over previous
import functools

import numpy as np

import jax
import jax.numpy as jnp
from jax import lax
from jax.experimental import pallas as pl
from jax.experimental.pallas import tpu as pltpu
from jax.experimental.pallas import tpu_sc as plsc

D_MODEL = 768
D_FF = 1024
N_EXPERTS = 64
CAPACITY = 80
AUX_COEF = 0.01
ZLOSS_COEF = 0.001
EPS = 1e-9

N_TOKENS = 4096
BLK_T = 512
N_BLK = N_TOKENS // BLK_T
DUMP = N_EXPERTS * CAPACITY
SLOTS = (N_EXPERTS + 1) * CAPACITY

SC_NC = 2
SC_NS = 16
SC_NW = SC_NC * SC_NS
TOK_PER_W = N_TOKENS // SC_NW


_TRIL = np.tril(np.ones((BLK_T, BLK_T), np.float32), -1)


def _router_body(l_ref, tril_ref, pos_ref, scale_ref, load_ref, imp_ref,
                 aux_ref, carry, imp_acc, z2_acc):
    i = pl.program_id(0)

    @pl.when(i == 0)
    def _init():
        carry[...] = jnp.zeros_like(carry)
        imp_acc[...] = jnp.zeros_like(imp_acc)
        z2_acc[0, 0] = 0.0

    logits = l_ref[...]
    m = jnp.max(logits, axis=1, keepdims=True)
    ex = jnp.exp(logits - m)
    s = jnp.sum(ex, axis=1, keepdims=True)
    probs = ex / s
    v = jnp.max(probs, axis=1, keepdims=True)
    iota_e = lax.broadcasted_iota(jnp.int32, (BLK_T, N_EXPERTS), 1)
    idx = jnp.min(jnp.where(probs == v, iota_e, N_EXPERTS), axis=1,
                  keepdims=True)
    oh = (iota_e == idx).astype(jnp.float32)

    prev = lax.dot_general(
        tril_ref[...], oh, (((1,), (0,)), ((), ())),
        preferred_element_type=jnp.float32,
        precision=lax.Precision.DEFAULT)
    rank_f = jnp.sum(oh * (prev + carry[0:1, :]), axis=1, keepdims=True)
    rank = rank_f.astype(jnp.int32)
    keep = rank < CAPACITY
    pos_ref[...] = jnp.where(keep, idx * CAPACITY + rank, DUMP)
    gate = v / (v + EPS)
    scale = jnp.where(keep, gate, 0.0)
    scale_ref[...] = jnp.broadcast_to(scale, (BLK_T, 128))

    carry[...] = carry[...] + jnp.sum(oh, axis=0, keepdims=True)
    imp_acc[...] = imp_acc[...] + jnp.sum(probs, axis=0, keepdims=True)
    z = m[:, 0] + jnp.log(s[:, 0])
    z2_acc[0, 0] = z2_acc[0, 0] + jnp.sum(z * z)

    @pl.when(i == N_BLK - 1)
    def _fin():
        cnt = carry[...]
        load = jnp.minimum(cnt, float(CAPACITY))
        load_ref[...] = load.astype(jnp.int32)
        imp = imp_acc[...]
        imp_ref[...] = imp
        imp_norm = imp[0:1, :] / (jnp.sum(imp[0:1, :]) + EPS)
        load_norm = load[0:1, :] / (jnp.sum(load[0:1, :]) + EPS)
        balance = N_EXPERTS * jnp.sum(imp_norm * load_norm)
        zloss = z2_acc[0, 0] / N_TOKENS
        aux = AUX_COEF * balance + ZLOSS_COEF * zloss
        aux_ref[...] = jnp.full((8, 128), aux, dtype=jnp.float32)


def _router(logits):
    return pl.pallas_call(
        _router_body,
        grid=(N_BLK,),
        in_specs=[
            pl.BlockSpec((BLK_T, N_EXPERTS), lambda i: (i, 0)),
            pl.BlockSpec((BLK_T, BLK_T), lambda i: (0, 0)),
        ],
        out_specs=[
            pl.BlockSpec((BLK_T, 1), lambda i: (i, 0)),
            pl.BlockSpec((BLK_T, 128), lambda i: (i, 0)),
            pl.BlockSpec((8, N_EXPERTS), lambda i: (0, 0)),
            pl.BlockSpec((8, N_EXPERTS), lambda i: (0, 0)),
            pl.BlockSpec((8, 128), lambda i: (0, 0)),
        ],
        out_shape=[
            jax.ShapeDtypeStruct((N_TOKENS, 1), jnp.int32),
            jax.ShapeDtypeStruct((N_TOKENS, 128), jnp.float32),
            jax.ShapeDtypeStruct((8, N_EXPERTS), jnp.int32),
            jax.ShapeDtypeStruct((8, N_EXPERTS), jnp.float32),
            jax.ShapeDtypeStruct((8, 128), jnp.float32),
        ],
        scratch_shapes=[
            pltpu.VMEM((8, N_EXPERTS), jnp.float32),
            pltpu.VMEM((8, N_EXPERTS), jnp.float32),
            pltpu.SMEM((1, 1), jnp.float32),
        ],
    )(logits, jnp.asarray(_TRIL))


def _dispatch(x_flat, pos, scale_exp):
    mesh = plsc.VectorSubcoreMesh(core_axis_name="c", subcore_axis_name="s")

    @functools.partial(
        pl.kernel, mesh=mesh,
        out_type=[
            jax.ShapeDtypeStruct((SLOTS, D_MODEL), jnp.float32),
            jax.ShapeDtypeStruct((SLOTS, 128), jnp.float32),
        ],
        scratch_types=[
            pltpu.VMEM((TOK_PER_W,), jnp.int32),
            pltpu.VMEM((TOK_PER_W, D_MODEL), jnp.float32),
            pltpu.VMEM((TOK_PER_W, 128), jnp.float32),
            pltpu.SemaphoreType.DMA,
            pltpu.SemaphoreType.DMA,
        ],
    )
    def disp(x_hbm, pos_hbm, se_hbm, xbuf_hbm, sbuf_hbm,
             idx_v, xrows_v, srows_v, sem1, sem2):
        wid = lax.axis_index("s") * SC_NC + lax.axis_index("c")
        base = wid * TOK_PER_W
        pltpu.sync_copy(pos_hbm.at[pl.ds(base, TOK_PER_W)], idx_v)
        pltpu.sync_copy(x_hbm.at[pl.ds(base, TOK_PER_W)], xrows_v)
        pltpu.sync_copy(se_hbm.at[pl.ds(base, TOK_PER_W)], srows_v)
        cp1 = pltpu.async_copy(xrows_v, xbuf_hbm.at[idx_v], sem1)
        cp2 = pltpu.async_copy(srows_v, sbuf_hbm.at[idx_v], sem2)
        cp1.wait()
        cp2.wait()

    return disp(x_flat, pos, scale_exp)


def _ffn_body(x_ref, w1_ref, b1_ref, w2_ref, b2_ref, s_ref, o_ref):
    xin = x_ref[...]
    h = jnp.dot(xin, w1_ref[0], preferred_element_type=jnp.float32,
                precision=lax.Precision.DEFAULT) + b1_ref[0]
    h = 0.5 * h * (1.0 + lax.erf(h * 0.7071067811865476))
    out = jnp.dot(h, w2_ref[0], preferred_element_type=jnp.float32,
                  precision=lax.Precision.DEFAULT) + b2_ref[0]
    s = s_ref[...][:, 0:1]
    o_ref[...] = jnp.where(s > 0.0, out * s, 0.0)


def _ffn(xbuf, sbuf, w1, b1, w2, b2):
    grid = N_EXPERTS + 1
    return pl.pallas_call(
        _ffn_body,
        grid=(grid,),
        in_specs=[
            pl.BlockSpec((CAPACITY, D_MODEL), lambda e: (e, 0)),
            pl.BlockSpec((1, D_MODEL, D_FF),
                         lambda e: (jnp.minimum(e, N_EXPERTS - 1), 0, 0)),
            pl.BlockSpec((1, 1, D_FF),
                         lambda e: (jnp.minimum(e, N_EXPERTS - 1), 0, 0)),
            pl.BlockSpec((1, D_FF, D_MODEL),
                         lambda e: (jnp.minimum(e, N_EXPERTS - 1), 0, 0)),
            pl.BlockSpec((1, 1, D_MODEL),
                         lambda e: (jnp.minimum(e, N_EXPERTS - 1), 0, 0)),
            pl.BlockSpec((CAPACITY, 128), lambda e: (e, 0)),
        ],
        out_specs=pl.BlockSpec((CAPACITY, D_MODEL), lambda e: (e, 0)),
        out_shape=jax.ShapeDtypeStruct((SLOTS, D_MODEL), jnp.float32),
    )(xbuf, w1, b1.reshape(N_EXPERTS, 1, D_FF), w2,
      b2.reshape(N_EXPERTS, 1, D_MODEL), sbuf)


def _combine(obuf, pos):
    mesh = plsc.VectorSubcoreMesh(core_axis_name="c", subcore_axis_name="s")

    @functools.partial(
        pl.kernel, mesh=mesh,
        out_type=jax.ShapeDtypeStruct((N_TOKENS, D_MODEL), jnp.float32),
        scratch_types=[
            pltpu.VMEM((TOK_PER_W,), jnp.int32),
            pltpu.VMEM((TOK_PER_W, D_MODEL), jnp.float32),
            pltpu.SemaphoreType.DMA,
        ],
    )
    def comb(obuf_hbm, pos_hbm, y_hbm, idx_v, rows_v, sem):
        wid = lax.axis_index("s") * SC_NC + lax.axis_index("c")
        base = wid * TOK_PER_W
        pltpu.sync_copy(pos_hbm.at[pl.ds(base, TOK_PER_W)], idx_v)
        pltpu.async_copy(obuf_hbm.at[idx_v], rows_v, sem).wait()
        pltpu.sync_copy(rows_v, y_hbm.at[pl.ds(base, TOK_PER_W)])

    return comb(obuf, pos)


def kernel(x, router_w, w1, b1, w2, b2):
    B, T, D = x.shape
    x_flat = x.reshape(B * T, D)
    logits = x_flat @ router_w.T
    pos2, scale_exp, load8, imp8, aux8 = _router(logits)
    pos = pos2.reshape(B * T)
    xbuf, sbuf = _dispatch(x_flat, pos, scale_exp)
    obuf = _ffn(xbuf, sbuf, w1, b1, w2, b2)
    y_flat = _combine(obuf, pos)
    y = y_flat.reshape(B, T, D)
    return (y, aux8[0, 0], load8[0], imp8[0])
